# Initial kernel scaffold; baseline (speedup 1.0000x reference)
import jax
import jax.numpy as jnp
from jax.experimental import pallas as pl


def kernel(indices, values, dense_mat):
    raise NotImplementedError("write your pallas kernel here")


# speedup vs baseline: 3.8339x; 3.8339x over previous
import functools

import jax
import jax.numpy as jnp
from jax import lax
from jax.experimental import pallas as pl
from jax.experimental.pallas import tpu as pltpu
from jax.experimental.pallas import tpu_sc as plsc

L = 16
NC = 2
NS = 16
B = 128
RS = 2048
NQ = 4


@functools.lru_cache(maxsize=None)
def _build(n, d, slice_len):
    qd = d // NQ
    n_per_sc = n // NC
    rows_per_tile = n_per_sc // NS
    qg = qd // L
    cap = slice_len + 2 * B

    mesh = plsc.VectorSubcoreMesh(core_axis_name="c", subcore_axis_name="s")

    @functools.partial(
        pl.kernel,
        out_type=jax.ShapeDtypeStruct((n, d), jnp.float32),
        mesh=mesh,
        compiler_params=pltpu.CompilerParams(
            needs_layout_passes=False, use_tc_tiling_on_sc=False),
        scratch_types=[
            pltpu.VMEM((RS,), jnp.int32),
            pltpu.VMEM((RS,), jnp.int32),
            pltpu.VMEM((RS,), jnp.float32),
            pltpu.VMEM((cap,), jnp.int32),
            pltpu.VMEM((cap,), jnp.float32),
            pltpu.VMEM((cap,), jnp.int32),
            pltpu.VMEM((B,), jnp.int32),
            pltpu.VMEM((B,), jnp.int32),
            pltpu.VMEM((B, qd), jnp.float32),
            pltpu.VMEM((B, qd), jnp.float32),
            pltpu.VMEM((L, qd), jnp.float32),
            pltpu.VMEM_SHARED((n_per_sc, qd), jnp.float32),
            pltpu.SemaphoreType.DMA,
            pltpu.SemaphoreType.DMA,
            pltpu.SemaphoreType.DMA,
            pltpu.SemaphoreType.DMA,
            pltpu.SemaphoreType.DMA,
        ],
    )
    def spmm(rows_hbm, cols_hbm, vals_hbm, d0_hbm, d1_hbm, d2_hbm, d3_hbm,
             out_hbm, rstage, cstage, vstage, ccol, cval, lrow,
             sidx0, sidx1, gbuf0, gbuf1, zbuf, acc,
             sem_l, sem_g0, sem_g1, sem_s0, sem_s1):
        c = lax.axis_index("c")
        s = lax.axis_index("s")

        fzero = jnp.zeros((L,), jnp.float32)
        for j in range(L):
            for g in range(qg):
                zbuf[j, pl.ds(g * L, L)] = fzero

        lanes = lax.iota(jnp.int32, L)
        izero = jnp.zeros((L,), jnp.int32)
        row0_v = jnp.broadcast_to(c * n_per_sc, (L,))
        npsc_v = jnp.full((L,), n_per_sc, jnp.int32)

        total = jnp.int32(0)
        for blk in range(slice_len // RS):
            sb = s * slice_len + blk * RS
            cp1 = pltpu.async_copy(rows_hbm.at[pl.ds(sb, RS)], rstage, sem_l)
            cp2 = pltpu.async_copy(cols_hbm.at[pl.ds(sb, RS)], cstage, sem_l)
            cp3 = pltpu.async_copy(vals_hbm.at[pl.ds(sb, RS)], vstage, sem_l)
            cp1.wait()
            cp2.wait()
            cp3.wait()

            def comp_body(i, off):
                r = rstage[pl.ds(i * L, L)]
                lr = r - row0_v
                m = (lr >= izero) & (lr < npsc_v)
                mi = m.astype(jnp.int32)
                inc = plsc.cumsum(mi)
                pos = (inc - mi) + jnp.broadcast_to(off, (L,))
                plsc.store_scatter(ccol, [pos], cstage[pl.ds(i * L, L)],
                                   mask=m)
                plsc.store_scatter(cval, [pos], vstage[pl.ds(i * L, L)],
                                   mask=m)
                plsc.store_scatter(lrow, [pos], lr, mask=m)
                return off + jnp.sum(mi)

            total = lax.fori_loop(0, RS // L, comp_body, total, unroll=2)

        pad_pos = jnp.broadcast_to(total, (L,)) + lanes
        tmask = jnp.ones((L,), jnp.bool_)
        for k in range(2 * B // L):
            kp = pad_pos + jnp.full((L,), k * L, jnp.int32)
            plsc.store_scatter(ccol, [kp], izero, mask=tmask)
            plsc.store_scatter(cval, [kp], fzero, mask=tmask)
            plsc.store_scatter(lrow, [kp], izero, mask=tmask)

        npairs = jnp.maximum((total + (2 * B - 1)) // (2 * B), 1)

        def stage_sidx(sidx_s, eb):
            for k in range(B // L):
                sidx_s[pl.ds(k * L, L)] = lrow[pl.ds(eb + k * L, L)]

        def scale(gbuf_s, eb):
            def scale_body(j, _):
                vsp = plsc.load_gather(
                    cval, [jnp.broadcast_to(eb + j, (L,))])
                for g in range(qg):
                    gbuf_s[j, pl.ds(g * L, L)] = (
                        gbuf_s[j, pl.ds(g * L, L)] * vsp)
                return 0

            lax.fori_loop(0, B, scale_body, 0, unroll=4)

        for dq, qi in ((d0_hbm, 0), (d1_hbm, 1), (d2_hbm, 2), (d3_hbm, 3)):
            for k in range(rows_per_tile // L):
                pltpu.sync_copy(
                    zbuf, acc.at[pl.ds(s * rows_per_tile + k * L, L)])

            stage_sidx(sidx0, 0)
            pltpu.async_copy(dq.at[ccol.at[pl.ds(0, B)]], gbuf0, sem_g0)
            plsc.subcore_barrier()

            def drain(gbuf_s, sem):
                pltpu.make_async_copy(dq.at[pl.ds(0, B)], gbuf_s, sem).wait()

            def pair_body(p, _):
                b0 = 2 * p
                eb0 = b0 * B
                eb1 = eb0 + B

                @pl.when(p > 0)
                def _():
                    drain(gbuf1, sem_s1)

                stage_sidx(sidx1, eb1)
                pltpu.async_copy(dq.at[ccol.at[pl.ds(eb1, B)]], gbuf1,
                                 sem_g1)

                drain(gbuf0, sem_g0)
                scale(gbuf0, eb0)
                pltpu.async_copy(gbuf0, acc.at[sidx0], sem_s0, add=True)

                drain(gbuf1, sem_g1)
                scale(gbuf1, eb1)
                pltpu.async_copy(gbuf1, acc.at[sidx1], sem_s1, add=True)

                drain(gbuf0, sem_s0)

                @pl.when(p + 1 < npairs)
                def _():
                    stage_sidx(sidx0, eb1 + B)
                    pltpu.async_copy(dq.at[ccol.at[pl.ds(eb1 + B, B)]],
                                     gbuf0, sem_g0)
                return 0

            lax.fori_loop(0, npairs, pair_body, 0)
            drain(gbuf1, sem_s1)

            plsc.subcore_barrier()
            pltpu.sync_copy(
                acc.at[pl.ds(s * rows_per_tile, rows_per_tile)],
                out_hbm.at[pl.ds(c * n_per_sc + s * rows_per_tile,
                                 rows_per_tile),
                           pl.ds(qi * qd, qd)])
            plsc.subcore_barrier()

    return spmm


def kernel(indices, values, dense_mat):
    nnz = values.shape[0]
    n, d = dense_mat.shape
    assert d % (NQ * L) == 0 and n % (NC * NS * L) == 0

    slice_len = -(-nnz // (NS * RS)) * RS
    nnz_pad = NS * slice_len
    pad = nnz_pad - nnz

    rows = indices[0]
    cols = indices[1]
    rows_p = jnp.concatenate([rows, jnp.full((pad,), n, rows.dtype)])
    cols_p = jnp.concatenate([cols, jnp.zeros((pad,), cols.dtype)])
    vals_p = jnp.concatenate([values, jnp.zeros((pad,), values.dtype)])
    qd = d // NQ
    dqs = [dense_mat[:, i * qd:(i + 1) * qd] for i in range(NQ)]

    f = _build(n, d, slice_len)
    return f(rows_p, cols_p, vals_p, *dqs)
